# Initial kernel scaffold; baseline (speedup 1.0000x reference)
import jax
import jax.numpy as jnp
from jax.experimental import pallas as pl


def kernel(timesteps, pe):
    raise NotImplementedError("write your pallas kernel here")


# speedup vs baseline: 6.7872x; 6.7872x over previous
import functools

import jax
import jax.numpy as jnp
from jax import lax
from jax.experimental import pallas as pl
from jax.experimental.pallas import tpu as pltpu
from jax.experimental.pallas import tpu_sc as plsc

_D = 64
_NC = 2
_NS = 16
_NW = _NC * _NS
_C = 1024
_G = 128


@functools.partial(jax.jit, static_argnums=(2,))
def _gather(idx_flat, table, total):
    b_per_w = total // _NW
    n_chunks = b_per_w // _C

    @functools.partial(
        pl.kernel,
        out_type=jax.ShapeDtypeStruct((total, _D), jnp.float32),
        mesh=plsc.VectorSubcoreMesh(core_axis_name="c", subcore_axis_name="s"),
        scratch_types=[
            pltpu.VMEM((_C,), jnp.int32),
            pltpu.VMEM((_C, _D), jnp.float32),
            pltpu.SemaphoreType.DMA,
        ],
        compiler_params=pltpu.CompilerParams(use_tc_tiling_on_sc=False),
    )
    def body(idx_hbm, table_hbm, out_hbm, idx_v, rows_v, sem):
        wid = lax.axis_index("s") * _NC + lax.axis_index("c")
        base = wid * b_per_w

        def chunk(g, carry):
            off = pl.multiple_of(base + g * _C, _C)
            pltpu.sync_copy(idx_hbm.at[pl.ds(off, _C)], idx_v)
            cps = [
                pltpu.async_copy(
                    table_hbm.at[idx_v.at[pl.ds(j * _G, _G)]],
                    rows_v.at[pl.ds(j * _G, _G)],
                    sem,
                )
                for j in range(_C // _G)
            ]
            for cp in cps:
                cp.wait()
            pltpu.sync_copy(rows_v, out_hbm.at[pl.ds(off, _C)])
            return carry

        lax.fori_loop(0, n_chunks, chunk, 0)

    return body(idx_flat, table)


def kernel(timesteps, pe):
    b, h = timesteps.shape
    flat = timesteps.reshape(-1)
    out = _gather(flat, pe, b * h)
    return out.reshape(b, h, pe.shape[1])


# speedup vs baseline: 6.9556x; 1.0248x over previous
import functools

import jax
import jax.numpy as jnp
from jax import lax
from jax.experimental import pallas as pl
from jax.experimental.pallas import tpu as pltpu
from jax.experimental.pallas import tpu_sc as plsc

_D = 64
_NC = 2
_NS = 16
_NW = _NC * _NS
_C = 640
_G = 128
_NB = 2


@functools.partial(jax.jit, static_argnums=(2,))
def _gather(idx_flat, table, total):
    b_per_w = total // _NW
    n_chunks = b_per_w // _C

    @functools.partial(
        pl.kernel,
        out_type=jax.ShapeDtypeStruct((total, _D), jnp.float32),
        mesh=plsc.VectorSubcoreMesh(core_axis_name="c", subcore_axis_name="s"),
        scratch_types=[
            pltpu.VMEM((b_per_w,), jnp.int32),
            pltpu.VMEM((_NB, _C, _D), jnp.float32),
            pltpu.SemaphoreType.DMA,
            pltpu.SemaphoreType.DMA,
        ],
        compiler_params=pltpu.CompilerParams(use_tc_tiling_on_sc=False),
    )
    def body(idx_hbm, table_hbm, out_hbm, idx_v, rows_v, gsem, ssem):
        wid = lax.axis_index("s") * _NC + lax.axis_index("c")
        base = wid * b_per_w
        pltpu.sync_copy(idx_hbm.at[pl.ds(base, b_per_w)], idx_v)

        def issue_gathers(g, b):
            ioff = pl.multiple_of(g * _C, 8)
            return [
                pltpu.async_copy(
                    table_hbm.at[idx_v.at[pl.ds(ioff + j * _G, _G)]],
                    rows_v.at[b].at[pl.ds(j * _G, _G)],
                    gsem,
                )
                for j in range(_C // _G)
            ]

        def issue_store(g, b):
            off = pl.multiple_of(base + g * _C, 8)
            pltpu.async_copy(rows_v.at[b], out_hbm.at[pl.ds(off, _C)], ssem)

        def drain_store():
            pltpu.make_async_copy(
                rows_v.at[0], out_hbm.at[pl.ds(base, _C)], ssem
            ).wait()

        for b in range(_NB):
            for cp in issue_gathers(b, b):
                cp.wait()
            issue_store(b, b)

        def steady(t, carry):
            for b in range(_NB):
                g = _NB * t + b
                drain_store()
                for cp in issue_gathers(g, b):
                    cp.wait()
                issue_store(g, b)
            return carry

        lax.fori_loop(1, n_chunks // _NB, steady, 0)

        for _ in range(_NB):
            drain_store()

    return body(idx_flat, table)


def kernel(timesteps, pe):
    b, h = timesteps.shape
    flat = timesteps.reshape(-1)
    out = _gather(flat, pe, b * h)
    return out.reshape(b, h, pe.shape[1])
